# Initial kernel scaffold; baseline (speedup 1.0000x reference)
import jax
import jax.numpy as jnp
from jax.experimental import pallas as pl


def kernel(x, edge_attr, u, edge_index):
    raise NotImplementedError("write your pallas kernel here")


# speedup vs baseline: 5.1341x; 5.1341x over previous
import functools

import jax
import jax.numpy as jnp
from jax import lax
from jax.experimental import pallas as pl
from jax.experimental.pallas import tpu as pltpu
from jax.experimental.pallas import tpu_sc as plsc

N_NODES = 10000
N_EDGES = 320000
D_EDGE = 16
D_FEAT = 128
D_GLOBAL = 128

NUM_CORES = 2
NUM_SUBCORES = 16
NW = NUM_CORES * NUM_SUBCORES
EPW = N_EDGES // NW
C = 125
K = 8
ROWS = N_EDGES // C
RPW = ROWS // NW
MEGA = RPW // K
STRIPE = 624
TAIL = N_NODES - NUM_SUBCORES * STRIPE


def _sc_segment_sum(dst2d, edge3d):
  mesh = plsc.VectorSubcoreMesh(core_axis_name="c", subcore_axis_name="s")

  @functools.partial(
      pl.kernel,
      out_type=(
          jax.ShapeDtypeStruct((NUM_CORES, N_NODES, D_EDGE), jnp.float32),
          jax.ShapeDtypeStruct((NUM_CORES, N_NODES, D_EDGE), jnp.float32),
      ),
      mesh=mesh,
      compiler_params=pltpu.CompilerParams(use_tc_tiling_on_sc=False),
      scratch_types=[
          pltpu.VMEM((K, C), jnp.int32),
          pltpu.VMEM((K, C, D_EDGE), jnp.float32),
          pltpu.VMEM((C, D_EDGE), jnp.float32),
          pltpu.VMEM((STRIPE, D_EDGE), jnp.float32),
          pltpu.VMEM_SHARED((N_NODES, D_EDGE), jnp.float32),
          pltpu.VMEM_SHARED((N_NODES, D_EDGE), jnp.float32),
      ],
  )
  def k(dst_hbm, edge_hbm, sums_hbm, counts_hbm, idx_v, rows_v, ones_v,
        zeros_v, sum_s, cnt_s):
    c = lax.axis_index("c")
    s = lax.axis_index("s")
    w = c * NUM_SUBCORES + s

    @pl.loop(0, STRIPE)
    def _(i):
      zeros_v[i, :] = jnp.zeros((16,), jnp.float32)

    @pl.loop(0, C)
    def _(i):
      ones_v[i, :] = jnp.ones((16,), jnp.float32)

    base_row = s * STRIPE
    pltpu.sync_copy(zeros_v, sum_s.at[pl.ds(base_row, STRIPE)])
    pltpu.sync_copy(zeros_v, cnt_s.at[pl.ds(base_row, STRIPE)])

    @pl.when(s == 0)
    def _():
      pltpu.sync_copy(zeros_v.at[pl.ds(0, TAIL)],
                      sum_s.at[pl.ds(NUM_SUBCORES * STRIPE, TAIL)])
      pltpu.sync_copy(zeros_v.at[pl.ds(0, TAIL)],
                      cnt_s.at[pl.ds(NUM_SUBCORES * STRIPE, TAIL)])

    plsc.subcore_barrier()

    @pl.loop(0, MEGA)
    def _(m):
      row0 = w * RPW + m * K
      pltpu.sync_copy(dst_hbm.at[pl.ds(row0, K), :], idx_v)
      pltpu.sync_copy(edge_hbm.at[pl.ds(row0, K)], rows_v)

      @pl.loop(0, K)
      def _(j):
        idx = idx_v.at[j]
        pltpu.sync_copy(rows_v.at[j], sum_s.at[idx], add=True)
        pltpu.sync_copy(ones_v, cnt_s.at[idx], add=True)

    plsc.subcore_barrier()

    pltpu.sync_copy(sum_s.at[pl.ds(base_row, STRIPE)],
                    sums_hbm.at[c, pl.ds(base_row, STRIPE)])
    pltpu.sync_copy(cnt_s.at[pl.ds(base_row, STRIPE)],
                    counts_hbm.at[c, pl.ds(base_row, STRIPE)])

    @pl.when(s == 0)
    def _():
      pltpu.sync_copy(sum_s.at[pl.ds(NUM_SUBCORES * STRIPE, TAIL)],
                      sums_hbm.at[c, pl.ds(NUM_SUBCORES * STRIPE, TAIL)])
      pltpu.sync_copy(cnt_s.at[pl.ds(NUM_SUBCORES * STRIPE, TAIL)],
                      counts_hbm.at[c, pl.ds(NUM_SUBCORES * STRIPE, TAIL)])

  return k(dst2d, edge3d)


_B = 1000


def _tc_finish_body(x_ref, u_ref, s_ref, c_ref, o_ref):
  total = s_ref[0] + s_ref[1]
  cnt = c_ref[0] + c_ref[1]
  agg = total / jnp.maximum(cnt, 1.0)
  u_b = jnp.broadcast_to(u_ref[...], (_B, D_GLOBAL))
  o_ref[...] = jnp.concatenate([agg, x_ref[...], u_b], axis=1)


def _tc_finish(x, u2d, sums, counts):
  grid = N_NODES // _B
  return pl.pallas_call(
      _tc_finish_body,
      grid=(grid,),
      in_specs=[
          pl.BlockSpec((_B, D_FEAT), lambda i: (i, 0)),
          pl.BlockSpec((1, D_GLOBAL), lambda i: (0, 0)),
          pl.BlockSpec((NUM_CORES, _B, D_EDGE), lambda i: (0, i, 0)),
          pl.BlockSpec((NUM_CORES, _B, D_EDGE), lambda i: (0, i, 0)),
      ],
      out_specs=pl.BlockSpec((_B, D_EDGE + D_FEAT + D_GLOBAL),
                             lambda i: (i, 0)),
      out_shape=jax.ShapeDtypeStruct(
          (N_NODES, D_EDGE + D_FEAT + D_GLOBAL), jnp.float32),
  )(x, u2d, sums, counts)


@jax.jit
def kernel(x, edge_attr, u, edge_index):
  dst2d = edge_index[1].astype(jnp.int32).reshape(ROWS, C)
  edge3d = edge_attr.reshape(ROWS, C, D_EDGE)
  sums, counts = _sc_segment_sum(dst2d, edge3d)
  return _tc_finish(x, u.reshape(1, D_GLOBAL), sums, counts)


# speedup vs baseline: 6.0094x; 1.1705x over previous
import functools

import jax
import jax.numpy as jnp
from jax import lax
from jax.experimental import pallas as pl
from jax.experimental.pallas import tpu as pltpu
from jax.experimental.pallas import tpu_sc as plsc

N_NODES = 10000
N_EDGES = 320000
D_EDGE = 16
D_FEAT = 128
D_GLOBAL = 128

NUM_CORES = 2
NUM_SUBCORES = 16
NW = NUM_CORES * NUM_SUBCORES
EPW = N_EDGES // NW
C = 80
K = 25
MEGA = EPW // (K * C)
CHUNK = K * C
STRIPE = 624
TAIL = N_NODES - NUM_SUBCORES * STRIPE


def _sc_segment_sum(dst, edge_attr):
  mesh = plsc.VectorSubcoreMesh(core_axis_name="c", subcore_axis_name="s")

  @functools.partial(
      pl.kernel,
      out_type=(
          jax.ShapeDtypeStruct((NUM_CORES, N_NODES, D_EDGE), jnp.float32),
          jax.ShapeDtypeStruct((NUM_CORES, N_NODES, D_EDGE), jnp.float32),
      ),
      mesh=mesh,
      compiler_params=pltpu.CompilerParams(use_tc_tiling_on_sc=False),
      scratch_types=[
          pltpu.VMEM((EPW,), jnp.int32),
          pltpu.VMEM((CHUNK, D_EDGE), jnp.float32),
          pltpu.VMEM((CHUNK, D_EDGE), jnp.float32),
          pltpu.VMEM((C, D_EDGE), jnp.float32),
          pltpu.VMEM((STRIPE, D_EDGE), jnp.float32),
          pltpu.VMEM_SHARED((N_NODES, D_EDGE), jnp.float32),
          pltpu.VMEM_SHARED((N_NODES, D_EDGE), jnp.float32),
          pltpu.SemaphoreType.DMA,
          pltpu.SemaphoreType.DMA,
          pltpu.SemaphoreType.DMA,
      ],
  )
  def k(dst_hbm, edge_hbm, sums_hbm, counts_hbm, idx_v, rows_a, rows_b,
        ones_v, zeros_v, sum_s, cnt_s, sem_i, sem_a, sem_b):
    c = lax.axis_index("c")
    s = lax.axis_index("s")
    w = c * NUM_SUBCORES + s
    e0 = w * EPW

    idx_dma = pltpu.async_copy(dst_hbm.at[1, pl.ds(e0, EPW)], idx_v, sem_i)

    @pl.loop(0, STRIPE)
    def _(i):
      zeros_v[i, :] = jnp.zeros((16,), jnp.float32)

    @pl.loop(0, C)
    def _(i):
      ones_v[i, :] = jnp.ones((16,), jnp.float32)

    base_row = s * STRIPE
    pltpu.sync_copy(zeros_v, sum_s.at[pl.ds(base_row, STRIPE)])
    pltpu.sync_copy(zeros_v, cnt_s.at[pl.ds(base_row, STRIPE)])

    @pl.when(s == 0)
    def _():
      pltpu.sync_copy(zeros_v.at[pl.ds(0, TAIL)],
                      sum_s.at[pl.ds(NUM_SUBCORES * STRIPE, TAIL)])
      pltpu.sync_copy(zeros_v.at[pl.ds(0, TAIL)],
                      cnt_s.at[pl.ds(NUM_SUBCORES * STRIPE, TAIL)])

    idx_dma.wait()
    plsc.subcore_barrier()

    bufs = (rows_a, rows_b)
    sems = (sem_a, sem_b)
    chunk_dmas = [None] * MEGA

    def start(m):
      chunk_dmas[m] = pltpu.async_copy(
          edge_hbm.at[pl.ds(e0 + m * CHUNK, CHUNK), :], bufs[m % 2],
          sems[m % 2])

    start(0)
    for m in range(MEGA):
      if m + 1 < MEGA:
        start(m + 1)
      chunk_dmas[m].wait()
      rows = bufs[m % 2]

      @pl.loop(0, K)
      def _(j, m=m, rows=rows):
        idx = idx_v.at[pl.ds((m * K + j) * C, C)]
        pltpu.sync_copy(rows.at[pl.ds(j * C, C)], sum_s.at[idx], add=True)
        pltpu.sync_copy(ones_v, cnt_s.at[idx], add=True)

    plsc.subcore_barrier()

    pltpu.sync_copy(sum_s.at[pl.ds(base_row, STRIPE)],
                    sums_hbm.at[c, pl.ds(base_row, STRIPE)])
    pltpu.sync_copy(cnt_s.at[pl.ds(base_row, STRIPE)],
                    counts_hbm.at[c, pl.ds(base_row, STRIPE)])

    @pl.when(s == 0)
    def _():
      pltpu.sync_copy(sum_s.at[pl.ds(NUM_SUBCORES * STRIPE, TAIL)],
                      sums_hbm.at[c, pl.ds(NUM_SUBCORES * STRIPE, TAIL)])
      pltpu.sync_copy(cnt_s.at[pl.ds(NUM_SUBCORES * STRIPE, TAIL)],
                      counts_hbm.at[c, pl.ds(NUM_SUBCORES * STRIPE, TAIL)])

  return k(dst, edge_attr)


_B = 1000


def _tc_finish_body(x_ref, u_ref, s_ref, c_ref, o_ref):
  total = s_ref[0] + s_ref[1]
  cnt = c_ref[0] + c_ref[1]
  agg = total / jnp.maximum(cnt, 1.0)
  u_b = jnp.broadcast_to(u_ref[...], (_B, D_GLOBAL))
  o_ref[...] = jnp.concatenate([agg, x_ref[...], u_b], axis=1)


def _tc_finish(x, u2d, sums, counts):
  grid = N_NODES // _B
  return pl.pallas_call(
      _tc_finish_body,
      grid=(grid,),
      in_specs=[
          pl.BlockSpec((_B, D_FEAT), lambda i: (i, 0)),
          pl.BlockSpec((1, D_GLOBAL), lambda i: (0, 0)),
          pl.BlockSpec((NUM_CORES, _B, D_EDGE), lambda i: (0, i, 0)),
          pl.BlockSpec((NUM_CORES, _B, D_EDGE), lambda i: (0, i, 0)),
      ],
      out_specs=pl.BlockSpec((_B, D_EDGE + D_FEAT + D_GLOBAL),
                             lambda i: (i, 0)),
      out_shape=jax.ShapeDtypeStruct(
          (N_NODES, D_EDGE + D_FEAT + D_GLOBAL), jnp.float32),
  )(x, u2d, sums, counts)


@jax.jit
def kernel(x, edge_attr, u, edge_index):
  dst = edge_index.astype(jnp.int32)
  sums, counts = _sc_segment_sum(dst, edge_attr)
  return _tc_finish(x, u.reshape(1, D_GLOBAL), sums, counts)
